# baseline (speedup 1.0000x reference)
import functools

import jax
import jax.numpy as jnp
from jax import lax
from jax.experimental import pallas as pl
from jax.experimental.pallas import tpu as pltpu
from jax.experimental.pallas import tpu_sc as plsc

NC = 2
NS = 16
NW = NC * NS
IDX_PER_DMA = 128
GROUPS = 8
CHUNK = GROUPS * IDX_PER_DMA


@functools.partial(jax.jit, static_argnums=(0, 1, 2))
def _run(total, nemb, nchunks, x4, vf, W):
    per_w = total // NW
    mesh = plsc.VectorSubcoreMesh(
        core_axis_name="c", subcore_axis_name="s",
        num_cores=NC, num_subcores=NS)

    @functools.partial(
        pl.kernel,
        out_type=jax.ShapeDtypeStruct((total, nemb), jnp.float32),
        mesh=mesh,
        scratch_types=[
            pltpu.VMEM((GROUPS, IDX_PER_DMA), jnp.int32),
            pltpu.VMEM((CHUNK,), jnp.float32),
            pltpu.VMEM((CHUNK, nemb), jnp.float32),
            pltpu.SemaphoreType.DMA,
        ],
        compiler_params=pltpu.CompilerParams(use_tc_tiling_on_sc=False),
    )
    def emb_kernel(x_hbm, v_hbm, w_hbm, out_hbm, idx_b, v_b, rows_b, sem):
        wid = lax.axis_index("s") * NC + lax.axis_index("c")
        base = wid * per_w

        def chunk_body(c, carry):
            off = base + c * CHUNK
            pltpu.sync_copy(x_hbm.at[wid, c], idx_b)
            descs = []
            for j in range(GROUPS):
                descs.append(pltpu.async_copy(
                    w_hbm.at[idx_b.at[j]],
                    rows_b.at[pl.ds(j * IDX_PER_DMA, IDX_PER_DMA)],
                    sem))
            pltpu.sync_copy(v_hbm.at[pl.ds(off, CHUNK)], v_b)
            for d in descs:
                d.wait()

            def group_body(g, _):
                v_vec = v_b[pl.ds(g * 16, 16)]
                for j in range(16):
                    i = g * 16 + j
                    rows_b[i] = rows_b[i] * v_vec[j]
                return 0

            lax.fori_loop(0, CHUNK // 16, group_body, 0)
            pltpu.sync_copy(rows_b, out_hbm.at[pl.ds(off, CHUNK)])
            return carry

        lax.fori_loop(0, nchunks, chunk_body, 0)

    return emb_kernel(x4, vf, W)


def kernel(x, v, W):
    B, F = x.shape
    total = B * F
    nemb = W.shape[1]
    nchunks = total // (NW * CHUNK)
    x4 = x.reshape(NW, nchunks, GROUPS, IDX_PER_DMA)
    vf = v.reshape(total)
    out = _run(total, nemb, nchunks, x4, vf, W)
    return out.reshape(B, F, nemb)
